# baseline (speedup 1.0000x reference)
import jax
import jax.numpy as jnp
from jax.experimental import pallas as pl
from jax.experimental.pallas import tpu as pltpu

_B, _S, _F, _K = 4, 2048, 1024, 64
_BR = 512
_BF = 512
_NR = _S // _BR
_NG = _F // _BF
_MIN32 = -2147483648


def _cumsum2d(x):
    d = 1
    while d < 128:
        sh = jnp.concatenate(
            [jnp.zeros_like(x[:, :d]), x[:, :-d]], axis=1)
        x = x + sh
        d *= 2
    row_tot = x[:, -1:]
    d = 1
    pre = jnp.concatenate(
        [jnp.zeros_like(row_tot[:1]), row_tot[:-1]], axis=0)
    while d < 16:
        sh = jnp.concatenate(
            [jnp.zeros_like(pre[:d]), pre[:-d]], axis=0)
        pre = pre + sh
        d *= 2
    return x + pre


def _topk1(wm):
    s = jax.lax.bitcast_convert_type(wm, jnp.int32)
    k32 = jnp.where(s < 0, _MIN32 - s, s)
    fl = (jax.lax.broadcasted_iota(jnp.int32, (16, 128), 0) * 128
          + jax.lax.broadcasted_iota(jnp.int32, (16, 128), 1))

    jvec = jax.lax.broadcasted_iota(jnp.int32, (8, 1, 1), 0)
    prefix = jnp.full((1, 1), _MIN32, jnp.int32)
    for p in (29, 26, 23, 20, 17, 14, 11, 8, 5, 2, 0):
        thr = prefix[None] + jnp.left_shift(jvec, p)
        cnt = jnp.sum((k32[None] >= thr).astype(jnp.int32), axis=(1, 2),
                      keepdims=True)
        d = jnp.sum((cnt >= _K).astype(jnp.int32), axis=0) - 1
        prefix = prefix + jnp.left_shift(d, p)
    t = prefix

    maskgt = k32 > t
    active = k32 == t
    g = jnp.sum(maskgt.astype(jnp.int32))
    need = _K - g
    eqrank = _cumsum2d(active.astype(jnp.int32))
    sel = maskgt | (active & (eqrank <= need))
    cpos = _cumsum2d(sel.astype(jnp.int32)) - 1

    cq = jnp.where(sel, cpos, -1)
    kio = jax.lax.broadcasted_iota(jnp.int32, (_K, 1, 1), 0)
    ohT = cq[None] == kio
    vc = jnp.sum(jnp.where(ohT, wm[None], 0.0), axis=(1, 2),
                 keepdims=True)[:, :, 0]
    ic = jnp.sum(jnp.where(ohT, fl[None], 0), axis=(1, 2),
                 keepdims=True)[:, :, 0]

    kio_s = kio[:, :, 0]
    kio_l = jnp.reshape(kio_s, (1, _K))
    vcl = jnp.reshape(vc, (1, _K))
    gtm = vcl > vc
    eqm = (vcl == vc) & (kio_l < kio_s)
    r = jnp.sum((gtm | eqm).astype(jnp.int32), axis=1, keepdims=True)

    idx = jnp.sum(jnp.where(r == kio_l, ic, 0), axis=0, keepdims=True)
    return idx


def _body(w_ref, x_ref, out_ref, acc_ref, idx_ref):
    r = pl.program_id(1)

    @pl.when(r < _NR)
    def _sum():
        part = jnp.sum(w_ref[0], axis=0).reshape(16, 128)

        @pl.when(r == 0)
        def _init():
            acc_ref[...] = part

        @pl.when(r != 0)
        def _acc():
            acc_ref[...] = acc_ref[...] + part

    @pl.when(r == _NR - 1)
    def _topk():
        idx_ref[...] = _topk1(acc_ref[...])

    @pl.when(r >= _NR)
    def _gather():
        idx = idx_ref[...]
        onehot = (jax.lax.broadcasted_iota(jnp.int32, (_S, _K), 0) == idx
                  ).astype(jnp.float32)
        out_ref[0] = jnp.dot(x_ref[0], onehot,
                             preferred_element_type=jnp.float32)


def kernel(x, w):
    out = pl.pallas_call(
        _body,
        grid=(_B, _NR + _NG),
        in_specs=[
            pl.BlockSpec((1, _BR, _S),
                         lambda b, r: (b, jnp.minimum(r, _NR - 1), 0)),
            pl.BlockSpec((1, _BF, _S),
                         lambda b, r: (b, jnp.clip(r - _NR, 0, _NG - 1), 0)),
        ],
        out_specs=pl.BlockSpec((1, _BF, _K),
                               lambda b, r: (b, jnp.clip(r - _NR, 0,
                                                         _NG - 1), 0)),
        out_shape=jax.ShapeDtypeStruct((_B, _F, _K), jnp.float32),
        scratch_shapes=[pltpu.VMEM((16, 128), jnp.float32),
                        pltpu.VMEM((1, _K), jnp.int32)],
    )(w, x)
    return out


# speedup vs baseline: 1.4794x; 1.4794x over previous
import jax
import jax.numpy as jnp
from jax.experimental import pallas as pl
from jax.experimental.pallas import tpu as pltpu

_B, _S, _F, _K = 4, 2048, 1024, 64
_BR = 1024
_BF = 1024


def _cumsum_lanes(x):
    n = x.shape[-1]
    d = 1
    while d < n:
        shifted = jnp.concatenate(
            [jnp.zeros_like(x[..., :d]), x[..., :-d]], axis=-1)
        x = x + shifted
        d *= 2
    return x


def _topk_idx(wm):
    s = jax.lax.bitcast_convert_type(wm, jnp.int32)
    k32 = jnp.where(s < 0, jnp.int32(-2147483648) - s, s)
    lane = jax.lax.broadcasted_iota(jnp.int32, (_B, _S), 1)

    prefix = jnp.full((_B, 1), jnp.int32(-2147483648))
    for bit in range(31, -1, -1):
        cand = prefix + jnp.int32(1 << bit if bit < 31 else -2147483648)
        cnt = jnp.sum((k32 >= cand).astype(jnp.int32), axis=1,
                      keepdims=True)
        prefix = jnp.where(cnt >= _K, cand, prefix)
    t = prefix
    maskgt = k32 > t
    active = k32 == t
    g = jnp.sum(maskgt.astype(jnp.int32), axis=1, keepdims=True)

    need = _K - g
    eqrank = _cumsum_lanes(active.astype(jnp.int32))
    sel = maskgt | (active & (eqrank <= need))
    cpos = _cumsum_lanes(sel.astype(jnp.int32)) - 1

    kio_s = jax.lax.broadcasted_iota(jnp.int32, (_B, _K, 1), 1)
    ohT = ((cpos[:, None, :] == kio_s) & sel[:, None, :])
    vc = jnp.sum(jnp.where(ohT, wm[:, None, :], 0.0), axis=2,
                 keepdims=True)
    ic = jnp.sum(jnp.where(ohT, lane[:, None, :], 0), axis=2,
                 keepdims=True)

    vcl = vc.reshape(_B, 1, _K)
    icl = ic.reshape(_B, 1, _K)
    gtm = vcl > vc
    eqm = (vcl == vc) & (icl < ic)
    r = jnp.sum((gtm | eqm).astype(jnp.int32), axis=2, keepdims=True)

    kio_l = jax.lax.broadcasted_iota(jnp.int32, (_B, 1, _K), 2)
    idx = jnp.sum(jnp.where(r == kio_l, ic, 0), axis=1)
    return idx


def _mean_topk_body(w_ref, idx_ref, acc_ref):
    b = pl.program_id(0)
    r = pl.program_id(1)
    nr = pl.num_programs(1)
    part = jnp.sum(w_ref[0], axis=0, keepdims=True)

    @pl.when(r == 0)
    def _init():
        acc_ref[pl.ds(b, 1), :] = part

    @pl.when(r != 0)
    def _acc():
        acc_ref[pl.ds(b, 1), :] = acc_ref[pl.ds(b, 1), :] + part

    @pl.when((b == _B - 1) & (r == nr - 1))
    def _topk():
        idx_ref[:, 0, :] = _topk_idx(acc_ref[...])


def _gather_body(idx_ref, x_ref, out_ref):
    idx = idx_ref[0]
    onehot = (jax.lax.broadcasted_iota(jnp.int32, (_S, _K), 0) == idx
              ).astype(jnp.float32)
    out_ref[0] = jnp.dot(x_ref[0], onehot,
                         preferred_element_type=jnp.float32)


def kernel(x, w):
    idx3 = pl.pallas_call(
        _mean_topk_body,
        grid=(_B, _S // _BR),
        in_specs=[pl.BlockSpec((1, _BR, _S), lambda b, r: (b, r, 0))],
        out_specs=pl.BlockSpec((_B, 1, _K), lambda b, r: (0, 0, 0)),
        out_shape=jax.ShapeDtypeStruct((_B, 1, _K), jnp.int32),
        scratch_shapes=[pltpu.VMEM((_B, _S), jnp.float32)],
    )(w)

    out = pl.pallas_call(
        _gather_body,
        grid=(_B, _F // _BF),
        in_specs=[
            pl.BlockSpec((1, 1, _K), lambda b, f: (b, 0, 0)),
            pl.BlockSpec((1, _BF, _S), lambda b, f: (b, f, 0)),
        ],
        out_specs=pl.BlockSpec((1, _BF, _K), lambda b, f: (b, f, 0)),
        out_shape=jax.ShapeDtypeStruct((_B, _F, _K), jnp.float32),
    )(idx3, x)
    return out


# speedup vs baseline: 1.5388x; 1.0402x over previous
import jax
import jax.numpy as jnp
from jax.experimental import pallas as pl
from jax.experimental.pallas import tpu as pltpu

_B, _S, _F, _K = 4, 2048, 1024, 64
_BR = 1024
_BF = 1024


def _cumsum_lanes(x):
    n = x.shape[-1]
    d = 1
    while d < n:
        shifted = jnp.concatenate(
            [jnp.zeros_like(x[..., :d]), x[..., :-d]], axis=-1)
        x = x + shifted
        d *= 2
    return x


def _topk_idx(wm):
    s = jax.lax.bitcast_convert_type(wm, jnp.int32)
    k32 = jnp.where(s < 0, jnp.int32(-2147483648) - s, s)
    lane = jax.lax.broadcasted_iota(jnp.int32, (_B, _S), 1)

    jvec = jax.lax.broadcasted_iota(jnp.int32, (_B, 8, 1), 1)
    prefix = jnp.full((_B, 1), jnp.int32(-2147483648))
    for p in (29, 26, 23, 20, 17, 14, 11, 8, 5, 2, 0):
        thr = prefix[:, :, None] + jnp.left_shift(jvec, p)
        cnt = jnp.sum((k32[:, None, :] >= thr).astype(jnp.int32),
                      axis=2, keepdims=True)
        d = jnp.sum((cnt >= _K).astype(jnp.int32), axis=1) - 1
        prefix = prefix + jnp.left_shift(d, p)
    t = prefix
    maskgt = k32 > t
    active = k32 == t
    g = jnp.sum(maskgt.astype(jnp.int32), axis=1, keepdims=True)

    need = _K - g
    eqrank = _cumsum_lanes(active.astype(jnp.int32))
    sel = maskgt | (active & (eqrank <= need))
    cpos = _cumsum_lanes(sel.astype(jnp.int32)) - 1

    kio_s = jax.lax.broadcasted_iota(jnp.int32, (_B, _K, 1), 1)
    ohT = ((cpos[:, None, :] == kio_s) & sel[:, None, :])
    vc = jnp.sum(jnp.where(ohT, wm[:, None, :], 0.0), axis=2,
                 keepdims=True)
    ic = jnp.sum(jnp.where(ohT, lane[:, None, :], 0), axis=2,
                 keepdims=True)

    vcl = vc.reshape(_B, 1, _K)
    icl = ic.reshape(_B, 1, _K)
    gtm = vcl > vc
    eqm = (vcl == vc) & (icl < ic)
    r = jnp.sum((gtm | eqm).astype(jnp.int32), axis=2, keepdims=True)

    kio_l = jax.lax.broadcasted_iota(jnp.int32, (_B, 1, _K), 2)
    idx = jnp.sum(jnp.where(r == kio_l, ic, 0), axis=1)
    return idx


def _mean_topk_body(w_ref, idx_ref, acc_ref):
    b = pl.program_id(0)
    r = pl.program_id(1)
    nr = pl.num_programs(1)
    part = jnp.sum(w_ref[0], axis=0, keepdims=True)

    @pl.when(r == 0)
    def _init():
        acc_ref[pl.ds(b, 1), :] = part

    @pl.when(r != 0)
    def _acc():
        acc_ref[pl.ds(b, 1), :] = acc_ref[pl.ds(b, 1), :] + part

    @pl.when((b == _B - 1) & (r == nr - 1))
    def _topk():
        idx_ref[:, 0, :] = _topk_idx(acc_ref[...])


def _gather_body(idx_ref, x_ref, out_ref):
    idx = idx_ref[0]
    onehot = (jax.lax.broadcasted_iota(jnp.int32, (_S, _K), 0) == idx
              ).astype(jnp.float32)
    out_ref[0] = jnp.dot(x_ref[0], onehot,
                         preferred_element_type=jnp.float32)


def kernel(x, w):
    idx3 = pl.pallas_call(
        _mean_topk_body,
        grid=(_B, _S // _BR),
        in_specs=[pl.BlockSpec((1, _BR, _S), lambda b, r: (b, r, 0))],
        out_specs=pl.BlockSpec((_B, 1, _K), lambda b, r: (0, 0, 0)),
        out_shape=jax.ShapeDtypeStruct((_B, 1, _K), jnp.int32),
        scratch_shapes=[pltpu.VMEM((_B, _S), jnp.float32)],
    )(w)

    out = pl.pallas_call(
        _gather_body,
        grid=(_B, _F // _BF),
        in_specs=[
            pl.BlockSpec((1, 1, _K), lambda b, f: (b, 0, 0)),
            pl.BlockSpec((1, _BF, _S), lambda b, f: (b, f, 0)),
        ],
        out_specs=pl.BlockSpec((1, _BF, _K), lambda b, f: (b, f, 0)),
        out_shape=jax.ShapeDtypeStruct((_B, _F, _K), jnp.float32),
    )(idx3, x)
    return out
